# baseline (speedup 1.0000x reference)
import functools

import jax
import jax.numpy as jnp
from jax import lax
from jax.experimental import pallas as pl
from jax.experimental.pallas import tpu as pltpu
from jax.experimental.pallas import tpu_sc as plsc

B = 4
IM = 128
L = 16
HENV = 256
WENV = 512
NPIX = IM * IM
NTEX = HENV * WENV
TBW = 16
NTILES = 32
PIX_PER_TILE = NPIX // NTILES
TEX_PER_TILE = NTEX // NTILES
PIXELS = 12000

_MESH = plsc.VectorSubcoreMesh(core_axis_name="c", subcore_axis_name="s",
                               num_cores=2, num_subcores=16)
_PARAMS = pltpu.CompilerParams(needs_layout_passes=False,
                               use_tc_tiling_on_sc=False)


def _tid():
    return lax.axis_index("s") * 2 + lax.axis_index("c")


def _iota16():
    return lax.iota(jnp.int32, 16)


def _splat(v):
    return jnp.full((16,), v, jnp.int32)


@functools.partial(
    pl.kernel,
    out_type=jax.ShapeDtypeStruct((NTEX, TBW), jnp.int32),
    mesh=_MESH,
    scratch_types=[
        pltpu.VMEM((2, 8, 3, 4, 128), jnp.float32),
        pltpu.VMEM((2, 512, TBW), jnp.int32),
        pltpu.SemaphoreType.DMA,
        pltpu.SemaphoreType.DMA,
        pltpu.SemaphoreType.DMA,
        pltpu.SemaphoreType.DMA,
    ],
    compiler_params=_PARAMS,
)
def _build_table(xr, yr, table, inb, outb, semi0, semi1, semo0, semo1):
    tid = _tid()
    t0 = tid * TEX_PER_TILE
    iota = _iota16()
    cols = [_splat(j) for j in range(12)]
    bufv = [_splat(0), _splat(1)]

    def issue_in(ch, buf, sem):
        for s in range(8):
            src = xr if s < 4 else yr
            pltpu.async_copy(src.at[s % 4, :, tid, :, ch, :],
                             inb.at[buf, s], sem)

    def drain_in(ch, buf, sem):
        for s in range(8):
            src = xr if s < 4 else yr
            pltpu.make_async_copy(src.at[s % 4, :, tid, :, ch, :],
                                  inb.at[buf, s], sem).wait()

    def issue_out(ch, buf, sem):
        base = t0 + ch * 512
        pltpu.async_copy(outb.at[buf], table.at[pl.ds(base, 512)], sem)

    def drain_out(ch, buf, sem):
        base = t0 + ch * 512
        pltpu.make_async_copy(outb.at[buf], table.at[pl.ds(base, 512)],
                              sem).wait()

    def shuffle(buf):
        @plsc.parallel_loop(0, 32, unroll=2)
        def grp(g):
            off = g * 16
            ridx = off + iota
            wt = g // 8
            l0 = (g - wt * 8) * 16
            v = [inb[buf, j // 3, j % 3, wt, pl.ds(l0, 16)]
                 for j in range(24)]
            for k in range(12):
                packed = plsc.bitcast(
                    plsc.pack(v[2 * k], v[2 * k + 1],
                              format=plsc.PackFormat.INTERLEAVED),
                    jnp.int32)
                cidx = (cols[k] + ridx) & 15
                plsc.store_scatter(outb, [bufv[buf], ridx, cidx], packed)

    issue_in(0, 0, semi0)

    def pair(j, _):
        ch0 = 2 * j
        ch1 = ch0 + 1
        drain_in(ch0, 0, semi0)
        issue_in(ch1, 1, semi1)

        @pl.when(j > 0)
        def _():
            drain_out(ch0 - 2, 0, semo0)

        shuffle(0)
        issue_out(ch0, 0, semo0)
        drain_in(ch1, 1, semi1)

        @pl.when(j < 3)
        def _():
            issue_in(ch0 + 2, 0, semi0)

        @pl.when(j > 0)
        def _():
            drain_out(ch1 - 2, 1, semo1)

        shuffle(1)
        issue_out(ch1, 1, semo1)
        return 0

    lax.fori_loop(0, 4, pair, 0)
    drain_out(6, 0, semo0)
    drain_out(7, 1, semo1)


@functools.partial(
    pl.kernel,
    out_type=(
        jax.ShapeDtypeStruct((3, NPIX), jnp.float32),
        jax.ShapeDtypeStruct((3, NPIX), jnp.float32),
        jax.ShapeDtypeStruct((NTILES, 16), jnp.float32),
    ),
    mesh=_MESH,
    scratch_types=[
        pltpu.VMEM((4, L, PIX_PER_TILE), jnp.int32),
        pltpu.VMEM((32, 4, 128), jnp.int32),
        pltpu.VMEM((2, 2, 3, PIX_PER_TILE), jnp.float32),
        pltpu.VMEM((2, PIX_PER_TILE, TBW), jnp.int32),
        pltpu.SemaphoreType.DMA,
        pltpu.SemaphoreType.DMA,
        pltpu.VMEM((5, 3, PIX_PER_TILE), jnp.float32),
        pltpu.VMEM((96,), jnp.float32),
        pltpu.VMEM((16,), jnp.float32),
    ],
    compiler_params=_PARAMS,
)
def _render(table, bs0, bd0, bs1, bd1, wflat, ix0, iy0, ix1, iy1,
            pred0, gt0, lpart,
            idxy, fid, brdfb, rows, sem0, sem1, acc, wv, lacc):
    tid = _tid()
    p0 = tid * PIX_PER_TILE

    pltpu.sync_copy(wflat, wv)
    pltpu.sync_copy(ix0.at[:, pl.ds(p0, PIX_PER_TILE)], idxy.at[0])
    pltpu.sync_copy(iy0.at[:, pl.ds(p0, PIX_PER_TILE)], idxy.at[1])
    pltpu.sync_copy(ix1.at[:, pl.ds(p0, PIX_PER_TILE)], idxy.at[2])
    pltpu.sync_copy(iy1.at[:, pl.ds(p0, PIX_PER_TILE)], idxy.at[3])

    def fid_l(l, i):
        il = i * 16 + l

        def fid_g(g, _):
            k = g // 8
            r = (g - k * 8) * 16
            off = g * 16
            vx = idxy[2 * i, l, pl.ds(off, 16)]
            vy = idxy[2 * i + 1, l, pl.ds(off, 16)]
            fid[il, k, pl.ds(r, 16)] = vy * WENV + vx
            return 0

        lax.fori_loop(0, 32, fid_g, 0)
        return i

    for i in range(2):
        lax.fori_loop(0, 16, fid_l, i)

    zero = jnp.zeros((16,), jnp.float32)

    def zg(g, pc):
        acc[pc // 3, pc % 3, pl.ds(g * 16, 16)] = zero
        return pc

    for plane in range(5):
        for c in range(3):
            lax.fori_loop(0, 32, zg, plane * 3 + c)
    lacc[pl.ds(0, 16)] = zero

    iota = _iota16()
    cols = [_splat(j) for j in range(12)]

    def issue(i, bs, bd, l, buf, sem):
        il = i * 16 + l
        for k in range(4):
            pltpu.async_copy(table.at[fid.at[il, k]],
                             rows.at[buf, pl.ds(k * 128, 128)], sem)
        pltpu.async_copy(bs.at[l, :, pl.ds(p0, PIX_PER_TILE)],
                         brdfb.at[buf, 0], sem)
        pltpu.async_copy(bd.at[l, :, pl.ds(p0, PIX_PER_TILE)],
                         brdfb.at[buf, 1], sem)

    def drain(i, bs, bd, l, buf, sem):
        il = i * 16 + l
        for k in range(4):
            pltpu.make_async_copy(table.at[fid.at[il, k]],
                                  rows.at[buf, pl.ds(k * 128, 128)],
                                  sem).wait()
        pltpu.make_async_copy(bs.at[l, :, pl.ds(p0, PIX_PER_TILE)],
                              brdfb.at[buf, 0], sem).wait()
        pltpu.make_async_copy(bd.at[l, :, pl.ds(p0, PIX_PER_TILE)],
                              brdfb.at[buf, 1], sem).wait()

    def compute(i, l, buf):
        wbase = i * 48 + l * 3
        wsp = [plsc.load_gather(wv, [_splat(0) + (wbase + c)])
               for c in range(3)]
        bufv = _splat(buf)

        il = i * 16 + l

        @plsc.parallel_loop(0, 32, unroll=4)
        def grp(g):
            off = g * 16
            ridx = off + iota
            k = g // 8
            r = (g - k * 8) * 16
            rot = fid[il, k, pl.ds(r, 16)] & 15
            vals = [None] * 24
            for kk in range(12):
                pi = plsc.load_gather(rows,
                                      [bufv, ridx, (cols[kk] + rot) & 15])
                a, b16 = plsc.unpack(plsc.bitcast(pi, jnp.bfloat16),
                                     format=plsc.PackFormat.INTERLEAVED,
                                     preferred_element_type=jnp.float32)
                vals[2 * kk] = a
                vals[2 * kk + 1] = b16
            for c in range(3):
                sv = brdfb[buf, 0, c, pl.ds(off, 16)]
                dv = brdfb[buf, 1, c, pl.ds(off, 16)]
                coeff = (dv + 10.0 * sv) * wsp[c]
                plsc.addupdate(acc.at[0, c, pl.ds(off, 16)],
                               coeff * vals[c])
                plsc.addupdate(acc.at[1, c, pl.ds(off, 16)],
                               coeff * vals[12 + c])
                for b in range(1, 4):
                    plsc.addupdate(
                        acc.at[b + 1, c, pl.ds(off, 16)],
                        coeff * (vals[b * 3 + c] - vals[12 + b * 3 + c]))

    def phase(i, bs, bd):
        issue(i, bs, bd, 0, 0, sem0)

        def pair(j, _):
            l0 = 2 * j
            l1 = 2 * j + 1
            drain(i, bs, bd, l0, 0, sem0)
            issue(i, bs, bd, l1, 1, sem1)
            compute(i, l0, 0)
            drain(i, bs, bd, l1, 1, sem1)

            @pl.when(j < 7)
            def _():
                issue(i, bs, bd, l1 + 1, 0, sem0)

            compute(i, l1, 1)
            return 0

        lax.fori_loop(0, 8, pair, 0)

    phase(0, bs0, bd0)
    phase(1, bs1, bd1)

    for c in range(3):
        def lg(g, _, c=c):
            off = g * 16
            d0 = acc[0, c, pl.ds(off, 16)] - acc[1, c, pl.ds(off, 16)]
            s = d0 * d0
            for b in range(1, 4):
                d = acc[b + 1, c, pl.ds(off, 16)]
                s = s + d * d
            plsc.addupdate(lacc.at[pl.ds(0, 16)], s)
            return 0

        lax.fori_loop(0, 32, lg, 0)

    pltpu.sync_copy(acc.at[0], pred0.at[:, pl.ds(p0, PIX_PER_TILE)])
    pltpu.sync_copy(acc.at[1], gt0.at[:, pl.ds(p0, PIX_PER_TILE)])
    pltpu.sync_copy(lacc, lpart.at[tid])


def kernel(x, y, brdfSpec_0, brdfDiffuse_0, weight_0, brdfSpec_1,
           brdfDiffuse_1, weight_1, idx_0, idy_0, idx_1, idy_1):
    def tiled_view(e):
        return jnp.transpose(
            jnp.transpose(e, (0, 3, 1, 2)).reshape(B, 3, 32, 8, 4, 128),
            (0, 1, 2, 4, 3, 5))

    xr = tiled_view(x)
    yr = tiled_view(y)
    table = _build_table(xr, yr)

    wflat = jnp.concatenate([weight_0.reshape(-1), weight_1.reshape(-1)])
    pred0, gt0, lpart = _render(
        table,
        brdfSpec_0.reshape(L, 3, NPIX), brdfDiffuse_0.reshape(L, 3, NPIX),
        brdfSpec_1.reshape(L, 3, NPIX), brdfDiffuse_1.reshape(L, 3, NPIX),
        wflat,
        idx_0.reshape(L, NPIX), idy_0.reshape(L, NPIX),
        idx_1.reshape(L, NPIX), idy_1.reshape(L, NPIX),
    )
    loss = jnp.sum(lpart) / (PIXELS * B * 3)
    return loss, pred0.reshape(3, IM, IM), gt0.reshape(3, IM, IM)


# speedup vs baseline: 1.1730x; 1.1730x over previous
import functools

import jax
import jax.numpy as jnp
from jax import lax
from jax.experimental import pallas as pl
from jax.experimental.pallas import tpu as pltpu
from jax.experimental.pallas import tpu_sc as plsc

B = 4
IM = 128
L = 16
HENV = 256
WENV = 512
NPIX = IM * IM
NTEX = HENV * WENV
ROWW = 32
NTILES = 32
PIX_PER_TILE = NPIX // NTILES
TEX_PER_TILE = NTEX // NTILES
PIXELS = 12000

_MESH = plsc.VectorSubcoreMesh(core_axis_name="c", subcore_axis_name="s",
                               num_cores=2, num_subcores=16)
_PARAMS = pltpu.CompilerParams(needs_layout_passes=False,
                               use_tc_tiling_on_sc=False)


def _tid():
    return lax.axis_index("s") * 2 + lax.axis_index("c")


def _iota16():
    return lax.iota(jnp.int32, 16)


def _splat(v):
    return jnp.full((16,), v, jnp.int32)


@functools.partial(
    pl.kernel,
    out_type=jax.ShapeDtypeStruct((NTEX, ROWW), jnp.float32),
    mesh=_MESH,
    scratch_types=[
        pltpu.VMEM((2, 8, 3, 4, 128), jnp.float32),
        pltpu.VMEM((2, 512, ROWW), jnp.float32),
        pltpu.SemaphoreType.DMA,
        pltpu.SemaphoreType.DMA,
        pltpu.SemaphoreType.DMA,
        pltpu.SemaphoreType.DMA,
    ],
    compiler_params=_PARAMS,
)
def _build_table(xr, yr, table, inb, outb, semi0, semi1, semo0, semo1):
    tid = _tid()
    t0 = tid * TEX_PER_TILE
    iota = _iota16()
    cols = [_splat(j) for j in range(24)]
    bufv = [_splat(0), _splat(1)]

    def issue_in(ch, buf, sem):
        for s in range(8):
            src = xr if s < 4 else yr
            pltpu.async_copy(src.at[s % 4, :, tid, :, ch, :],
                             inb.at[buf, s], sem)

    def drain_in(ch, buf, sem):
        for s in range(8):
            src = xr if s < 4 else yr
            pltpu.make_async_copy(src.at[s % 4, :, tid, :, ch, :],
                                  inb.at[buf, s], sem).wait()

    def issue_out(ch, buf, sem):
        base = t0 + ch * 512
        pltpu.async_copy(outb.at[buf], table.at[pl.ds(base, 512)], sem)

    def drain_out(ch, buf, sem):
        base = t0 + ch * 512
        pltpu.make_async_copy(outb.at[buf], table.at[pl.ds(base, 512)],
                              sem).wait()

    def shuffle(buf):
        @plsc.parallel_loop(0, 32, unroll=2)
        def grp(g):
            off = g * 16
            ridx = off + iota
            wt = g // 8
            l0 = (g - wt * 8) * 16
            for s in range(8):
                for c in range(3):
                    v = inb[buf, s, c, wt, pl.ds(l0, 16)]
                    cidx = (cols[s * 3 + c] + ridx) & 31
                    plsc.store_scatter(outb, [bufv[buf], ridx, cidx], v)

    issue_in(0, 0, semi0)

    def pair(j, _):
        ch0 = 2 * j
        ch1 = ch0 + 1
        drain_in(ch0, 0, semi0)
        issue_in(ch1, 1, semi1)

        @pl.when(j > 0)
        def _():
            drain_out(ch0 - 2, 0, semo0)

        shuffle(0)
        issue_out(ch0, 0, semo0)
        drain_in(ch1, 1, semi1)

        @pl.when(j < 3)
        def _():
            issue_in(ch0 + 2, 0, semi0)

        @pl.when(j > 0)
        def _():
            drain_out(ch1 - 2, 1, semo1)

        shuffle(1)
        issue_out(ch1, 1, semo1)
        return 0

    lax.fori_loop(0, 4, pair, 0)
    drain_out(6, 0, semo0)
    drain_out(7, 1, semo1)


@functools.partial(
    pl.kernel,
    out_type=(
        jax.ShapeDtypeStruct((3, NPIX), jnp.float32),
        jax.ShapeDtypeStruct((3, NPIX), jnp.float32),
        jax.ShapeDtypeStruct((NTILES, 16), jnp.float32),
    ),
    mesh=_MESH,
    scratch_types=[
        pltpu.VMEM((4, L, PIX_PER_TILE), jnp.int32),
        pltpu.VMEM((32, 4, 128), jnp.int32),
        pltpu.VMEM((2, 2, 3, PIX_PER_TILE), jnp.float32),
        pltpu.VMEM((2, PIX_PER_TILE, ROWW), jnp.float32),
        pltpu.SemaphoreType.DMA,
        pltpu.SemaphoreType.DMA,
        pltpu.VMEM((5, 3, PIX_PER_TILE), jnp.float32),
        pltpu.VMEM((96,), jnp.float32),
        pltpu.VMEM((16,), jnp.float32),
    ],
    compiler_params=_PARAMS,
)
def _render(table, bs0, bd0, bs1, bd1, wflat, ix0, iy0, ix1, iy1,
            pred0, gt0, lpart,
            idxy, fid, brdfb, rows, sem0, sem1, acc, wv, lacc):
    tid = _tid()
    p0 = tid * PIX_PER_TILE

    pltpu.sync_copy(wflat, wv)
    pltpu.sync_copy(ix0.at[:, pl.ds(p0, PIX_PER_TILE)], idxy.at[0])
    pltpu.sync_copy(iy0.at[:, pl.ds(p0, PIX_PER_TILE)], idxy.at[1])
    pltpu.sync_copy(ix1.at[:, pl.ds(p0, PIX_PER_TILE)], idxy.at[2])
    pltpu.sync_copy(iy1.at[:, pl.ds(p0, PIX_PER_TILE)], idxy.at[3])

    def fid_l(l, i):
        il = i * 16 + l

        def fid_g(g, _):
            k = g // 8
            r = (g - k * 8) * 16
            off = g * 16
            vx = idxy[2 * i, l, pl.ds(off, 16)]
            vy = idxy[2 * i + 1, l, pl.ds(off, 16)]
            fid[il, k, pl.ds(r, 16)] = vy * WENV + vx
            return 0

        lax.fori_loop(0, 32, fid_g, 0)
        return i

    for i in range(2):
        lax.fori_loop(0, 16, fid_l, i)

    zero = jnp.zeros((16,), jnp.float32)

    def zg(g, pc):
        acc[pc // 3, pc % 3, pl.ds(g * 16, 16)] = zero
        return pc

    for plane in range(5):
        for c in range(3):
            lax.fori_loop(0, 32, zg, plane * 3 + c)
    lacc[pl.ds(0, 16)] = zero

    iota = _iota16()
    cols = [_splat(j) for j in range(24)]

    def issue(i, bs, bd, l, buf, sem):
        il = i * 16 + l
        for k in range(4):
            pltpu.async_copy(table.at[fid.at[il, k]],
                             rows.at[buf, pl.ds(k * 128, 128)], sem)
        pltpu.async_copy(bs.at[l, :, pl.ds(p0, PIX_PER_TILE)],
                         brdfb.at[buf, 0], sem)
        pltpu.async_copy(bd.at[l, :, pl.ds(p0, PIX_PER_TILE)],
                         brdfb.at[buf, 1], sem)

    def drain(i, bs, bd, l, buf, sem):
        il = i * 16 + l
        for k in range(4):
            pltpu.make_async_copy(table.at[fid.at[il, k]],
                                  rows.at[buf, pl.ds(k * 128, 128)],
                                  sem).wait()
        pltpu.make_async_copy(bs.at[l, :, pl.ds(p0, PIX_PER_TILE)],
                              brdfb.at[buf, 0], sem).wait()
        pltpu.make_async_copy(bd.at[l, :, pl.ds(p0, PIX_PER_TILE)],
                              brdfb.at[buf, 1], sem).wait()

    def compute(i, l, buf):
        wbase = i * 48 + l * 3
        wsp = [plsc.load_gather(wv, [_splat(0) + (wbase + c)])
               for c in range(3)]
        bufv = _splat(buf)

        il = i * 16 + l

        @plsc.parallel_loop(0, 32, unroll=4)
        def grp(g):
            off = g * 16
            ridx = off + iota
            k = g // 8
            r = (g - k * 8) * 16
            rot = fid[il, k, pl.ds(r, 16)] & 31
            for c in range(3):
                sv = brdfb[buf, 0, c, pl.ds(off, 16)]
                dv = brdfb[buf, 1, c, pl.ds(off, 16)]
                coeff = (dv + 10.0 * sv) * wsp[c]
                gx0 = plsc.load_gather(rows,
                                       [bufv, ridx, (cols[c] + rot) & 31])
                gy0 = plsc.load_gather(rows,
                                       [bufv, ridx, (cols[12 + c] + rot) & 31])
                plsc.addupdate(acc.at[0, c, pl.ds(off, 16)], coeff * gx0)
                plsc.addupdate(acc.at[1, c, pl.ds(off, 16)], coeff * gy0)
                for b in range(1, 4):
                    gx = plsc.load_gather(
                        rows, [bufv, ridx, (cols[b * 3 + c] + rot) & 31])
                    gy = plsc.load_gather(
                        rows, [bufv, ridx, (cols[12 + b * 3 + c] + rot) & 31])
                    plsc.addupdate(acc.at[b + 1, c, pl.ds(off, 16)],
                                   coeff * (gx - gy))

    def phase(i, bs, bd):
        issue(i, bs, bd, 0, 0, sem0)

        def pair(j, _):
            l0 = 2 * j
            l1 = 2 * j + 1
            drain(i, bs, bd, l0, 0, sem0)
            issue(i, bs, bd, l1, 1, sem1)
            compute(i, l0, 0)
            drain(i, bs, bd, l1, 1, sem1)

            @pl.when(j < 7)
            def _():
                issue(i, bs, bd, l1 + 1, 0, sem0)

            compute(i, l1, 1)
            return 0

        lax.fori_loop(0, 8, pair, 0)

    phase(0, bs0, bd0)
    phase(1, bs1, bd1)

    for c in range(3):
        def lg(g, _, c=c):
            off = g * 16
            d0 = acc[0, c, pl.ds(off, 16)] - acc[1, c, pl.ds(off, 16)]
            s = d0 * d0
            for b in range(1, 4):
                d = acc[b + 1, c, pl.ds(off, 16)]
                s = s + d * d
            plsc.addupdate(lacc.at[pl.ds(0, 16)], s)
            return 0

        lax.fori_loop(0, 32, lg, 0)

    pltpu.sync_copy(acc.at[0], pred0.at[:, pl.ds(p0, PIX_PER_TILE)])
    pltpu.sync_copy(acc.at[1], gt0.at[:, pl.ds(p0, PIX_PER_TILE)])
    pltpu.sync_copy(lacc, lpart.at[tid])


def kernel(x, y, brdfSpec_0, brdfDiffuse_0, weight_0, brdfSpec_1,
           brdfDiffuse_1, weight_1, idx_0, idy_0, idx_1, idy_1):
    def tiled_view(e):
        return jnp.transpose(
            jnp.transpose(e, (0, 3, 1, 2)).reshape(B, 3, 32, 8, 4, 128),
            (0, 1, 2, 4, 3, 5))

    xr = tiled_view(x)
    yr = tiled_view(y)
    table = _build_table(xr, yr)

    wflat = jnp.concatenate([weight_0.reshape(-1), weight_1.reshape(-1)])
    pred0, gt0, lpart = _render(
        table,
        brdfSpec_0.reshape(L, 3, NPIX), brdfDiffuse_0.reshape(L, 3, NPIX),
        brdfSpec_1.reshape(L, 3, NPIX), brdfDiffuse_1.reshape(L, 3, NPIX),
        wflat,
        idx_0.reshape(L, NPIX), idy_0.reshape(L, NPIX),
        idx_1.reshape(L, NPIX), idy_1.reshape(L, NPIX),
    )
    loss = jnp.sum(lpart) / (PIXELS * B * 3)
    return loss, pred0.reshape(3, IM, IM), gt0.reshape(3, IM, IM)
